# baseline (speedup 1.0000x reference)
import functools

import jax
import jax.numpy as jnp
from jax import lax
from jax.experimental import pallas as pl
from jax.experimental.pallas import tpu as pltpu
from jax.experimental.pallas import tpu_sc as plsc

N = 10000
D = 128
NC = 2
NS = 16
NW = NC * NS
NPAD = 10240
CH = 128


def _mesh():
    return plsc.VectorSubcoreMesh(core_axis_name="c", subcore_axis_name="s",
                                  num_cores=NC, num_subcores=NS)


def _make_deg_kernel(chunks):
    @functools.partial(
        pl.kernel,
        mesh=_mesh(),
        out_type=jax.ShapeDtypeStruct((NC, NPAD), jnp.float32),
        scratch_types=[
            pltpu.VMEM((chunks, CH), jnp.int32),
            pltpu.VMEM((CH,), jnp.float32),
            pltpu.VMEM_SHARED((NPAD,), jnp.float32),
        ],
    )
    def deg_kernel(dst_hbm, ones_hbm, zeros_hbm, out_hbm, dstv, onesv, cnt):
        c = lax.axis_index("c")
        s = lax.axis_index("s")
        w = s * NC + c
        pltpu.sync_copy(dst_hbm.at[w], dstv)
        pltpu.sync_copy(ones_hbm, onesv)

        @pl.when(s == 0)
        def _():
            pltpu.sync_copy(zeros_hbm, cnt)

        plsc.subcore_barrier()

        def body(j, carry):
            pltpu.sync_copy(onesv, cnt.at[dstv.at[j]], add=True)
            return carry

        lax.fori_loop(0, chunks, body, 0)
        plsc.subcore_barrier()

        @pl.when(s == 0)
        def _():
            pltpu.sync_copy(cnt, out_hbm.at[c])

    return deg_kernel


STRIPE = NPAD // NS
ZC = STRIPE // CH


def _make_agg_kernel(chunks):
    half = chunks // 2

    @functools.partial(
        pl.kernel,
        mesh=_mesh(),
        out_type=jax.ShapeDtypeStruct((NC, NPAD, D), jnp.float32),
        scratch_types=[
            pltpu.VMEM((half, CH), jnp.int32),
            pltpu.VMEM((half, CH), jnp.int32),
            pltpu.VMEM((ZC, CH), jnp.int32),
            pltpu.VMEM((CH, D), jnp.float32),
            pltpu.VMEM((CH, D), jnp.float32),
            pltpu.VMEM_SHARED((NPAD, D), jnp.float32),
            pltpu.SemaphoreType.DMA,
            pltpu.SemaphoreType.DMA,
            pltpu.SemaphoreType.DMA,
            pltpu.SemaphoreType.DMA,
        ],
    )
    def agg_kernel(src_hbm, dst_hbm, g_hbm, zeros_hbm, iota_hbm, out_hbm,
                   srcv, dstv, idxz, rows0, rows1, acc, sem0, sem1,
                   ssem0, ssem1):
        c = lax.axis_index("c")
        s = lax.axis_index("s")
        w = s * NC + c

        pltpu.sync_copy(iota_hbm.at[s], idxz)
        pltpu.sync_copy(zeros_hbm, rows0)
        for i in range(ZC):
            pltpu.sync_copy(rows0, acc.at[idxz.at[i]])

        plsc.subcore_barrier()

        rows = (rows0, rows1)
        sems = (sem0, sem1)
        ssems = (ssem0, ssem1)
        for h in range(2):
            pltpu.sync_copy(src_hbm.at[w, pl.ds(h * half, half)], srcv)
            pltpu.sync_copy(dst_hbm.at[w, pl.ds(h * half, half)], dstv)
            pltpu.async_copy(g_hbm.at[srcv.at[0]], rows0, sem0)

            def body(j, carry):
                for b in range(2):
                    k = j * 2 + b
                    nxt = k + 1

                    pltpu.make_async_copy(
                        g_hbm.at[srcv.at[k]], rows[b], sems[b]).wait()
                    pltpu.async_copy(
                        rows[b], acc.at[dstv.at[k]], ssems[b], add=True)

                    @pl.when(nxt < half)
                    def _():
                        @pl.when(k > 0)
                        def _():
                            pltpu.make_async_copy(
                                rows[1 - b], acc.at[dstv.at[k - 1]],
                                ssems[1 - b]).wait()
                        pltpu.async_copy(
                            g_hbm.at[srcv.at[nxt]], rows[1 - b], sems[1 - b])
                return carry

            lax.fori_loop(0, half // 2, body, 0)
            pltpu.make_async_copy(
                rows[1], acc.at[dstv.at[half - 1]], ssems[1]).wait()
        plsc.subcore_barrier()

        for i in range(ZC):
            pltpu.async_copy(acc.at[idxz.at[i]], rows[i % 2], sems[i % 2])
            pltpu.make_async_copy(
                acc.at[idxz.at[i]], rows[i % 2], sems[i % 2]).wait()
            pltpu.sync_copy(
                rows[i % 2],
                out_hbm.at[c, pl.ds(s * STRIPE + i * CH, CH)])

    return agg_kernel


RB = 1000


def _mm_scale_body(cnt_ref, x_ref, w_ref, dinv_ref, g_ref):
    deg = 1.0 + cnt_ref[0] + cnt_ref[1]
    dinv = lax.rsqrt(deg)
    dinv_ref[...] = dinv
    h = jnp.dot(x_ref[...], w_ref[...], preferred_element_type=jnp.float32)
    g_ref[...] = h * dinv


def _mm_scale(cnt, x, w):
    return pl.pallas_call(
        _mm_scale_body,
        grid=(N // RB,),
        in_specs=[
            pl.BlockSpec((NC, RB, 1), lambda i: (0, i, 0)),
            pl.BlockSpec((RB, D), lambda i: (i, 0)),
            pl.BlockSpec((D, D), lambda i: (0, 0)),
        ],
        out_specs=[
            pl.BlockSpec((RB, 1), lambda i: (i, 0)),
            pl.BlockSpec((RB, D), lambda i: (i, 0)),
        ],
        out_shape=[
            jax.ShapeDtypeStruct((N, 1), jnp.float32),
            jax.ShapeDtypeStruct((N, D), jnp.float32),
        ],
    )(cnt, x, w)


def _mid_body(a_ref, g_ref, dinv_ref, b_ref, w_ref, o_ref):
    agg = a_ref[0] + a_ref[1] + g_ref[...]
    t = agg * dinv_ref[...] + b_ref[...]
    t = jnp.where(t >= 0, t, 0.01 * t)
    h = jnp.dot(t, w_ref[...], preferred_element_type=jnp.float32)
    o_ref[...] = h * dinv_ref[...]


def _mid(a, g, dinv, b, w):
    return pl.pallas_call(
        _mid_body,
        grid=(N // RB,),
        in_specs=[
            pl.BlockSpec((NC, RB, D), lambda i: (0, i, 0)),
            pl.BlockSpec((RB, D), lambda i: (i, 0)),
            pl.BlockSpec((RB, 1), lambda i: (i, 0)),
            pl.BlockSpec((1, D), lambda i: (0, 0)),
            pl.BlockSpec((D, D), lambda i: (0, 0)),
        ],
        out_specs=pl.BlockSpec((RB, D), lambda i: (i, 0)),
        out_shape=jax.ShapeDtypeStruct((N, D), jnp.float32),
    )(a, g, dinv, b, w)


def _final_body(a_ref, g_ref, dinv_ref, b_ref, x_ref, o_ref):
    t = (a_ref[0] + a_ref[1] + g_ref[...]) * dinv_ref[...] + b_ref[...]
    t = jnp.where(t >= 0, t, 0.01 * t)
    o_ref[...] = t + x_ref[...]


def _final(a, g, dinv, b, x):
    return pl.pallas_call(
        _final_body,
        grid=(N // RB,),
        in_specs=[
            pl.BlockSpec((NC, RB, D), lambda i: (0, i, 0)),
            pl.BlockSpec((RB, D), lambda i: (i, 0)),
            pl.BlockSpec((RB, 1), lambda i: (i, 0)),
            pl.BlockSpec((1, D), lambda i: (0, 0)),
            pl.BlockSpec((RB, D), lambda i: (i, 0)),
        ],
        out_specs=pl.BlockSpec((RB, D), lambda i: (i, 0)),
        out_shape=jax.ShapeDtypeStruct((N, D), jnp.float32),
    )(a, g, dinv, b, x)


def kernel(x, edge_index, W0, b0, W1, b1):
    src = edge_index[0]
    dst = edge_index[1]
    e = src.shape[0]
    chunks = -(-e // (NW * CH))
    chunks += (-chunks) % 4
    ep = NW * CH * chunks
    pad = ep - e
    if pad:
        pad_ids = jnp.arange(pad, dtype=jnp.int32)
        src = jnp.concatenate([src, pad_ids % N])
        dst = jnp.concatenate([dst, N + pad_ids % (NPAD - N)])
    src3 = src.reshape(NW, chunks, CH)
    dst3 = dst.reshape(NW, chunks, CH)

    ones1 = jnp.ones((CH,), jnp.float32)
    zeros1 = jnp.zeros((NPAD,), jnp.float32)
    zrows = jnp.zeros((CH, D), jnp.float32)
    iota3 = jnp.arange(NPAD, dtype=jnp.int32).reshape(NS, ZC, CH)

    cnt = _make_deg_kernel(chunks)(dst3, ones1, zeros1)
    cnt = cnt.reshape(NC, NPAD, 1)
    dinv, g0 = _mm_scale(cnt, x, W0)

    agg = _make_agg_kernel(chunks)
    a0 = agg(src3, dst3, g0, zrows, iota3)
    g1 = _mid(a0, g0, dinv, b0.reshape(1, D), W1)
    a1 = agg(src3, dst3, g1, zrows, iota3)
    return _final(a1, g1, dinv, b1.reshape(1, D), x)


# speedup vs baseline: 1.1385x; 1.1385x over previous
import functools

import jax
import jax.numpy as jnp
from jax import lax
from jax.experimental import pallas as pl
from jax.experimental.pallas import tpu as pltpu
from jax.experimental.pallas import tpu_sc as plsc

N = 10000
D = 128
NC = 2
NS = 16
NW = NC * NS
NPAD = 10240
CH = 128


def _mesh():
    return plsc.VectorSubcoreMesh(core_axis_name="c", subcore_axis_name="s",
                                  num_cores=NC, num_subcores=NS)


def _make_deg_kernel(chunks):
    @functools.partial(
        pl.kernel,
        mesh=_mesh(),
        out_type=jax.ShapeDtypeStruct((NC, NPAD), jnp.float32),
        scratch_types=[
            pltpu.VMEM((chunks, CH), jnp.int32),
            pltpu.VMEM((CH,), jnp.float32),
            pltpu.VMEM_SHARED((NPAD,), jnp.float32),
        ],
    )
    def deg_kernel(dst_hbm, ones_hbm, zeros_hbm, out_hbm, dstv, onesv, cnt):
        c = lax.axis_index("c")
        s = lax.axis_index("s")
        w = s * NC + c
        pltpu.sync_copy(dst_hbm.at[w], dstv)
        pltpu.sync_copy(ones_hbm, onesv)

        @pl.when(s == 0)
        def _():
            pltpu.sync_copy(zeros_hbm, cnt)

        plsc.subcore_barrier()

        def body(j, carry):
            pltpu.sync_copy(onesv, cnt.at[dstv.at[j]], add=True)
            return carry

        lax.fori_loop(0, chunks, body, 0)
        plsc.subcore_barrier()

        @pl.when(s == 0)
        def _():
            pltpu.sync_copy(cnt, out_hbm.at[c])

    return deg_kernel


STRIPE = NPAD // NS
ZC = STRIPE // CH


def _make_agg_kernel(chunks):
    half = chunks // 2

    @functools.partial(
        pl.kernel,
        mesh=_mesh(),
        out_type=jax.ShapeDtypeStruct((NC, NPAD, D), jnp.float32),
        scratch_types=[
            pltpu.VMEM((half, CH), jnp.int32),
            pltpu.VMEM((half, CH), jnp.int32),
            pltpu.VMEM((ZC, CH), jnp.int32),
            pltpu.VMEM((CH, D), jnp.float32),
            pltpu.VMEM((CH, D), jnp.float32),
            pltpu.VMEM_SHARED((NPAD, D), jnp.float32),
            pltpu.SemaphoreType.DMA,
            pltpu.SemaphoreType.DMA,
        ],
    )
    def agg_kernel(src_hbm, dst_hbm, g_hbm, zeros_hbm, iota_hbm, out_hbm,
                   srcv, dstv, idxz, rows0, rows1, acc, sem0, sem1):
        c = lax.axis_index("c")
        s = lax.axis_index("s")
        w = s * NC + c

        pltpu.sync_copy(iota_hbm.at[s], idxz)
        pltpu.sync_copy(zeros_hbm, rows0)
        for i in range(ZC):
            pltpu.sync_copy(rows0, acc.at[idxz.at[i]])

        plsc.subcore_barrier()

        rows = (rows0, rows1)
        sems = (sem0, sem1)
        for h in range(2):
            pltpu.sync_copy(src_hbm.at[w, pl.ds(h * half, half)], srcv)
            pltpu.sync_copy(dst_hbm.at[w, pl.ds(h * half, half)], dstv)
            pltpu.async_copy(g_hbm.at[srcv.at[0]], rows0, sem0)

            def body(j, carry):
                for b in range(2):
                    k = j * 2 + b
                    nxt = k + 1

                    @pl.when(nxt < half)
                    def _():
                        pltpu.async_copy(
                            g_hbm.at[srcv.at[nxt]], rows[1 - b], sems[1 - b])

                    pltpu.make_async_copy(
                        g_hbm.at[srcv.at[k]], rows[b], sems[b]).wait()
                    pltpu.sync_copy(rows[b], acc.at[dstv.at[k]], add=True)
                return carry

            lax.fori_loop(0, half // 2, body, 0)
        plsc.subcore_barrier()

        for i in range(ZC):
            pltpu.async_copy(acc.at[idxz.at[i]], rows[i % 2], sems[i % 2])
            pltpu.make_async_copy(
                acc.at[idxz.at[i]], rows[i % 2], sems[i % 2]).wait()
            pltpu.sync_copy(
                rows[i % 2],
                out_hbm.at[c, pl.ds(s * STRIPE + i * CH, CH)])

    return agg_kernel


RB = 1000


def _mm_scale_body(cnt_ref, x_ref, w_ref, dinv_ref, g_ref):
    deg = 1.0 + cnt_ref[0] + cnt_ref[1]
    dinv = lax.rsqrt(deg)
    dinv_ref[...] = dinv
    h = jnp.dot(x_ref[...], w_ref[...], preferred_element_type=jnp.float32)
    g_ref[...] = h * dinv


def _mm_scale(cnt, x, w):
    return pl.pallas_call(
        _mm_scale_body,
        grid=(N // RB,),
        in_specs=[
            pl.BlockSpec((NC, RB, 1), lambda i: (0, i, 0)),
            pl.BlockSpec((RB, D), lambda i: (i, 0)),
            pl.BlockSpec((D, D), lambda i: (0, 0)),
        ],
        out_specs=[
            pl.BlockSpec((RB, 1), lambda i: (i, 0)),
            pl.BlockSpec((RB, D), lambda i: (i, 0)),
        ],
        out_shape=[
            jax.ShapeDtypeStruct((N, 1), jnp.float32),
            jax.ShapeDtypeStruct((N, D), jnp.float32),
        ],
    )(cnt, x, w)


def _mid_body(a_ref, g_ref, dinv_ref, b_ref, w_ref, o_ref):
    agg = a_ref[0] + a_ref[1] + g_ref[...]
    t = agg * dinv_ref[...] + b_ref[...]
    t = jnp.where(t >= 0, t, 0.01 * t)
    h = jnp.dot(t, w_ref[...], preferred_element_type=jnp.float32)
    o_ref[...] = h * dinv_ref[...]


def _mid(a, g, dinv, b, w):
    return pl.pallas_call(
        _mid_body,
        grid=(N // RB,),
        in_specs=[
            pl.BlockSpec((NC, RB, D), lambda i: (0, i, 0)),
            pl.BlockSpec((RB, D), lambda i: (i, 0)),
            pl.BlockSpec((RB, 1), lambda i: (i, 0)),
            pl.BlockSpec((1, D), lambda i: (0, 0)),
            pl.BlockSpec((D, D), lambda i: (0, 0)),
        ],
        out_specs=pl.BlockSpec((RB, D), lambda i: (i, 0)),
        out_shape=jax.ShapeDtypeStruct((N, D), jnp.float32),
    )(a, g, dinv, b, w)


def _final_body(a_ref, g_ref, dinv_ref, b_ref, x_ref, o_ref):
    t = (a_ref[0] + a_ref[1] + g_ref[...]) * dinv_ref[...] + b_ref[...]
    t = jnp.where(t >= 0, t, 0.01 * t)
    o_ref[...] = t + x_ref[...]


def _final(a, g, dinv, b, x):
    return pl.pallas_call(
        _final_body,
        grid=(N // RB,),
        in_specs=[
            pl.BlockSpec((NC, RB, D), lambda i: (0, i, 0)),
            pl.BlockSpec((RB, D), lambda i: (i, 0)),
            pl.BlockSpec((RB, 1), lambda i: (i, 0)),
            pl.BlockSpec((1, D), lambda i: (0, 0)),
            pl.BlockSpec((RB, D), lambda i: (i, 0)),
        ],
        out_specs=pl.BlockSpec((RB, D), lambda i: (i, 0)),
        out_shape=jax.ShapeDtypeStruct((N, D), jnp.float32),
    )(a, g, dinv, b, x)


def kernel(x, edge_index, W0, b0, W1, b1):
    src = edge_index[0]
    dst = edge_index[1]
    e = src.shape[0]
    chunks = -(-e // (NW * CH))
    chunks += (-chunks) % 4
    ep = NW * CH * chunks
    pad = ep - e
    if pad:
        pad_ids = jnp.arange(pad, dtype=jnp.int32)
        src = jnp.concatenate([src, pad_ids % N])
        dst = jnp.concatenate([dst, N + pad_ids % (NPAD - N)])
    src3 = src.reshape(NW, chunks, CH)
    dst3 = dst.reshape(NW, chunks, CH)

    ones1 = jnp.ones((CH,), jnp.float32)
    zeros1 = jnp.zeros((NPAD,), jnp.float32)
    zrows = jnp.zeros((CH, D), jnp.float32)
    iota3 = jnp.arange(NPAD, dtype=jnp.int32).reshape(NS, ZC, CH)

    cnt = _make_deg_kernel(chunks)(dst3, ones1, zeros1)
    cnt = cnt.reshape(NC, NPAD, 1)
    dinv, g0 = _mm_scale(cnt, x, W0)

    agg = _make_agg_kernel(chunks)
    a0 = agg(src3, dst3, g0, zrows, iota3)
    g1 = _mid(a0, g0, dinv, b0.reshape(1, D), W1)
    a1 = agg(src3, dst3, g1, zrows, iota3)
    return _final(a1, g1, dinv, b1.reshape(1, D), x)


# speedup vs baseline: 1.1569x; 1.0161x over previous
import functools

import jax
import jax.numpy as jnp
from jax import lax
from jax.experimental import pallas as pl
from jax.experimental.pallas import tpu as pltpu
from jax.experimental.pallas import tpu_sc as plsc

N = 10000
D = 128
NC = 2
NS = 16
NW = NC * NS
NPAD = 10240
CH = 128


def _mesh():
    return plsc.VectorSubcoreMesh(core_axis_name="c", subcore_axis_name="s",
                                  num_cores=NC, num_subcores=NS)


def _make_deg_kernel(chunks):
    @functools.partial(
        pl.kernel,
        mesh=_mesh(),
        out_type=jax.ShapeDtypeStruct((NC, NPAD), jnp.float32),
        scratch_types=[
            pltpu.VMEM((chunks, CH), jnp.int32),
            pltpu.VMEM((CH,), jnp.float32),
            pltpu.VMEM_SHARED((NPAD,), jnp.float32),
        ],
    )
    def deg_kernel(dst_hbm, ones_hbm, zeros_hbm, out_hbm, dstv, onesv, cnt):
        c = lax.axis_index("c")
        s = lax.axis_index("s")
        w = s * NC + c
        pltpu.sync_copy(dst_hbm.at[w], dstv)
        pltpu.sync_copy(ones_hbm, onesv)

        @pl.when(s == 0)
        def _():
            pltpu.sync_copy(zeros_hbm, cnt)

        plsc.subcore_barrier()

        def body(j, carry):
            pltpu.sync_copy(onesv, cnt.at[dstv.at[j]], add=True)
            return carry

        lax.fori_loop(0, chunks, body, 0)
        plsc.subcore_barrier()

        @pl.when(s == 0)
        def _():
            pltpu.sync_copy(cnt, out_hbm.at[c])

    return deg_kernel


STRIPE = NPAD // NS
ZC = STRIPE // CH


def _make_agg_kernel(chunks):
    half = chunks // 2

    @functools.partial(
        pl.kernel,
        mesh=_mesh(),
        out_type=jax.ShapeDtypeStruct((NC, NPAD, D), jnp.float32),
        scratch_types=[
            pltpu.VMEM((half, CH), jnp.int32),
            pltpu.VMEM((half, CH), jnp.int32),
            pltpu.VMEM((ZC, CH), jnp.int32),
            pltpu.VMEM((CH, D), jnp.float32),
            pltpu.VMEM((CH, D), jnp.float32),
            pltpu.VMEM_SHARED((NPAD, D), jnp.float32),
            pltpu.SemaphoreType.DMA,
            pltpu.SemaphoreType.DMA,
        ],
    )
    def agg_kernel(src_hbm, dst_hbm, g_hbm, zeros_hbm, iota_hbm, out_hbm,
                   srcv, dstv, idxz, rows0, rows1, acc, sem0, sem1):
        c = lax.axis_index("c")
        s = lax.axis_index("s")
        w = s * NC + c

        pltpu.sync_copy(iota_hbm.at[s], idxz)
        pltpu.sync_copy(zeros_hbm, rows0)
        for i in range(ZC):
            pltpu.sync_copy(rows0, acc.at[idxz.at[i]])

        plsc.subcore_barrier()

        rows = (rows0, rows1)
        sems = (sem0, sem1)
        for h in range(2):
            pltpu.sync_copy(src_hbm.at[w, pl.ds(h * half, half)], srcv)
            pltpu.sync_copy(dst_hbm.at[w, pl.ds(h * half, half)], dstv)
            pltpu.async_copy(g_hbm.at[srcv.at[0]], rows0, sem0)

            def body(j, carry):
                for b in range(2):
                    k = j * 2 + b
                    nxt = k + 1

                    @pl.when(nxt < half)
                    def _():
                        pltpu.async_copy(
                            g_hbm.at[srcv.at[nxt]], rows[1 - b], sems[1 - b])

                    pltpu.make_async_copy(
                        g_hbm.at[srcv.at[k]], rows[b], sems[b]).wait()
                    pltpu.sync_copy(rows[b], acc.at[dstv.at[k]], add=True)
                return carry

            lax.fori_loop(0, half // 2, body, 0)
        plsc.subcore_barrier()

        for i in range(ZC):
            pltpu.async_copy(acc.at[idxz.at[i]], rows[i % 2], sems[i % 2])
            pltpu.make_async_copy(
                acc.at[idxz.at[i]], rows[i % 2], sems[i % 2]).wait()
            pltpu.sync_copy(
                rows[i % 2],
                out_hbm.at[c, pl.ds(s * STRIPE + i * CH, CH)])

    return agg_kernel


RB = 2000


def _mm_scale_body(cnt_ref, x_ref, w_ref, dinv_ref, g_ref):
    deg = 1.0 + cnt_ref[0] + cnt_ref[1]
    dinv = lax.rsqrt(deg)
    dinv_ref[...] = dinv
    h = jnp.dot(x_ref[...], w_ref[...], preferred_element_type=jnp.float32)
    g_ref[...] = h * dinv


def _mm_scale(cnt, x, w):
    return pl.pallas_call(
        _mm_scale_body,
        grid=(N // RB,),
        in_specs=[
            pl.BlockSpec((NC, RB, 1), lambda i: (0, i, 0)),
            pl.BlockSpec((RB, D), lambda i: (i, 0)),
            pl.BlockSpec((D, D), lambda i: (0, 0)),
        ],
        out_specs=[
            pl.BlockSpec((RB, 1), lambda i: (i, 0)),
            pl.BlockSpec((RB, D), lambda i: (i, 0)),
        ],
        out_shape=[
            jax.ShapeDtypeStruct((N, 1), jnp.float32),
            jax.ShapeDtypeStruct((N, D), jnp.float32),
        ],
    )(cnt, x, w)


def _mid_body(a_ref, g_ref, dinv_ref, b_ref, w_ref, o_ref):
    agg = a_ref[0] + a_ref[1] + g_ref[...]
    t = agg * dinv_ref[...] + b_ref[...]
    t = jnp.where(t >= 0, t, 0.01 * t)
    h = jnp.dot(t, w_ref[...], preferred_element_type=jnp.float32)
    o_ref[...] = h * dinv_ref[...]


def _mid(a, g, dinv, b, w):
    return pl.pallas_call(
        _mid_body,
        grid=(N // RB,),
        in_specs=[
            pl.BlockSpec((NC, RB, D), lambda i: (0, i, 0)),
            pl.BlockSpec((RB, D), lambda i: (i, 0)),
            pl.BlockSpec((RB, 1), lambda i: (i, 0)),
            pl.BlockSpec((1, D), lambda i: (0, 0)),
            pl.BlockSpec((D, D), lambda i: (0, 0)),
        ],
        out_specs=pl.BlockSpec((RB, D), lambda i: (i, 0)),
        out_shape=jax.ShapeDtypeStruct((N, D), jnp.float32),
    )(a, g, dinv, b, w)


def _final_body(a_ref, g_ref, dinv_ref, b_ref, x_ref, o_ref):
    t = (a_ref[0] + a_ref[1] + g_ref[...]) * dinv_ref[...] + b_ref[...]
    t = jnp.where(t >= 0, t, 0.01 * t)
    o_ref[...] = t + x_ref[...]


def _final(a, g, dinv, b, x):
    return pl.pallas_call(
        _final_body,
        grid=(N // RB,),
        in_specs=[
            pl.BlockSpec((NC, RB, D), lambda i: (0, i, 0)),
            pl.BlockSpec((RB, D), lambda i: (i, 0)),
            pl.BlockSpec((RB, 1), lambda i: (i, 0)),
            pl.BlockSpec((1, D), lambda i: (0, 0)),
            pl.BlockSpec((RB, D), lambda i: (i, 0)),
        ],
        out_specs=pl.BlockSpec((RB, D), lambda i: (i, 0)),
        out_shape=jax.ShapeDtypeStruct((N, D), jnp.float32),
    )(a, g, dinv, b, x)


def kernel(x, edge_index, W0, b0, W1, b1):
    src = edge_index[0]
    dst = edge_index[1]
    e = src.shape[0]
    chunks = -(-e // (NW * CH))
    chunks += (-chunks) % 4
    ep = NW * CH * chunks
    pad = ep - e
    if pad:
        pad_ids = jnp.arange(pad, dtype=jnp.int32)
        src = jnp.concatenate([src, pad_ids % N])
        dst = jnp.concatenate([dst, N + pad_ids % (NPAD - N)])
    src3 = src.reshape(NW, chunks, CH)
    dst3 = dst.reshape(NW, chunks, CH)

    ones1 = jnp.ones((CH,), jnp.float32)
    zeros1 = jnp.zeros((NPAD,), jnp.float32)
    zrows = jnp.zeros((CH, D), jnp.float32)
    iota3 = jnp.arange(NPAD, dtype=jnp.int32).reshape(NS, ZC, CH)

    cnt = _make_deg_kernel(chunks)(dst3, ones1, zeros1)
    cnt = cnt.reshape(NC, NPAD, 1)
    dinv, g0 = _mm_scale(cnt, x, W0)

    agg = _make_agg_kernel(chunks)
    a0 = agg(src3, dst3, g0, zrows, iota3)
    g1 = _mid(a0, g0, dinv, b0.reshape(1, D), W1)
    a1 = agg(src3, dst3, g1, zrows, iota3)
    return _final(a1, g1, dinv, b1.reshape(1, D), x)
